# initial kernel scaffold (unmeasured)
import jax
import jax.numpy as jnp
from jax import lax
from jax.experimental import pallas as pl
from jax.experimental.pallas import tpu as pltpu


def kernel(
    x,
):
    def body(*refs):
        pass

    out_shape = jax.ShapeDtypeStruct(..., jnp.float32)
    return pl.pallas_call(body, out_shape=out_shape)(...)


# baseline (device time: 811578 ns/iter reference)
import jax
import jax.numpy as jnp
from jax import lax
from jax.experimental import pallas as pl
from jax.experimental.pallas import tpu as pltpu

N_DEV = 32


_YZ_PATH = [
    (0, 0), (1, 0), (2, 0), (3, 0),
    (3, 1), (2, 1), (1, 1), (0, 1),
    (0, 2), (1, 2), (2, 2), (3, 2),
    (3, 3), (2, 3), (1, 3), (0, 3),
]

_RING_COORDS = [(0, y, z) for (y, z) in _YZ_PATH] + [
    (1, y, z) for (y, z) in reversed(_YZ_PATH)
]


def _mesh_id(c):
    x, y, z = c
    xi = x if y % 2 == 0 else 1 - x
    return z * 8 + y * 2 + xi


_PERM = [_mesh_id(c) for c in _RING_COORDS]
_INV = [0] * N_DEV
for _r, _m in enumerate(_PERM):
    _INV[_m] = _r
assert sorted(_PERM) == list(range(N_DEV))


def kernel(x):
    m_per, n = x.shape

    perm = jnp.array(_PERM, dtype=jnp.int32)
    inv = jnp.array(_INV, dtype=jnp.int32)

    def body(perm_ref, inv_ref, x_ref, out_ref, own_ref, local_sem,
             send_sems, recv_sems):
        my_pos = lax.axis_index("i")
        r = inv_ref[my_pos]
        right = perm_ref[(r + 1) % N_DEV]
        left = perm_ref[(r - 1) % N_DEV]

        barrier_sem = pltpu.get_barrier_semaphore()
        for nbr in (left, right):
            pl.semaphore_signal(
                barrier_sem, inc=1,
                device_id=(nbr,), device_id_type=pl.DeviceIdType.MESH,
            )
        pl.semaphore_wait(barrier_sem, 2)

        own_ref[:, :] = x_ref[:, :].astype(jnp.bfloat16)
        cp = pltpu.make_async_copy(
            own_ref, out_ref.at[pl.ds(my_pos * m_per, m_per), :], local_sem
        )
        cp.start()
        cp.wait()

        for h in range(N_DEV - 1):
            origin = perm_ref[(r - h) % N_DEV]
            rdma = pltpu.make_async_remote_copy(
                src_ref=out_ref.at[pl.ds(origin * m_per, m_per), :],
                dst_ref=out_ref.at[pl.ds(origin * m_per, m_per), :],
                send_sem=send_sems.at[h],
                recv_sem=recv_sems.at[h],
                device_id=(right,),
                device_id_type=pl.DeviceIdType.MESH,
            )
            rdma.start()
            rdma.wait()

    return pl.pallas_call(
        body,
        out_shape=jax.ShapeDtypeStruct((N_DEV * m_per, n), jnp.bfloat16),
        grid_spec=pltpu.PrefetchScalarGridSpec(
            num_scalar_prefetch=2,
            in_specs=[pl.BlockSpec(memory_space=pltpu.VMEM)],
            out_specs=pl.BlockSpec(memory_space=pltpu.MemorySpace.HBM),
            scratch_shapes=[
                pltpu.VMEM((m_per, n), jnp.bfloat16),
                pltpu.SemaphoreType.DMA,
                pltpu.SemaphoreType.DMA((N_DEV - 1,)),
                pltpu.SemaphoreType.DMA((N_DEV - 1,)),
            ],
        ),
        compiler_params=pltpu.CompilerParams(collective_id=0),
    )(perm, inv, x)


# device time: 447448 ns/iter; 1.8138x vs baseline; 1.8138x over previous
import jax
import jax.numpy as jnp
from jax import lax
from jax.experimental import pallas as pl
from jax.experimental.pallas import tpu as pltpu

N_DEV = 32


_YZ_PATH = [
    (0, 0), (1, 0), (2, 0), (3, 0),
    (3, 1), (2, 1), (1, 1), (0, 1),
    (0, 2), (1, 2), (2, 2), (3, 2),
    (3, 3), (2, 3), (1, 3), (0, 3),
]

_RING_COORDS = [(0, y, z) for (y, z) in _YZ_PATH] + [
    (1, y, z) for (y, z) in reversed(_YZ_PATH)
]


def _mesh_id(c):
    x, y, z = c
    xi = x if y % 2 == 0 else 1 - x
    return z * 8 + y * 2 + xi


_PERM = [_mesh_id(c) for c in _RING_COORDS]
_INV = [0] * N_DEV
for _r, _m in enumerate(_PERM):
    _INV[_m] = _r
assert sorted(_PERM) == list(range(N_DEV))


def kernel(x):
    m_per, n = x.shape

    perm = jnp.array(_PERM, dtype=jnp.int32)
    inv = jnp.array(_INV, dtype=jnp.int32)

    def body(perm_ref, inv_ref, x_ref, out_ref, own_ref, local_sem,
             send_sems, recv_sems):
        my_pos = lax.axis_index("i")
        r = inv_ref[my_pos]
        right = perm_ref[(r + 1) % N_DEV]
        left = perm_ref[(r - 1) % N_DEV]

        barrier_sem = pltpu.get_barrier_semaphore()
        for nbr in (left, right):
            pl.semaphore_signal(
                barrier_sem, inc=1,
                device_id=(nbr,), device_id_type=pl.DeviceIdType.MESH,
            )
        pl.semaphore_wait(barrier_sem, 2)

        own_ref[:, :] = x_ref[:, :].astype(jnp.bfloat16)
        cp = pltpu.make_async_copy(
            own_ref, out_ref.at[pl.ds(my_pos * m_per, m_per), :], local_sem
        )
        cp.start()
        cp.wait()

        n_right = N_DEV // 2
        n_left = N_DEV - 1 - n_right
        for h in range(n_right):
            origin_r = perm_ref[(r - h) % N_DEV]
            rdma_r = pltpu.make_async_remote_copy(
                src_ref=out_ref.at[pl.ds(origin_r * m_per, m_per), :],
                dst_ref=out_ref.at[pl.ds(origin_r * m_per, m_per), :],
                send_sem=send_sems.at[h],
                recv_sem=recv_sems.at[h],
                device_id=(right,),
                device_id_type=pl.DeviceIdType.MESH,
            )
            rdma_r.start()
            if h < n_left:
                origin_l = perm_ref[(r + h) % N_DEV]
                rdma_l = pltpu.make_async_remote_copy(
                    src_ref=out_ref.at[pl.ds(origin_l * m_per, m_per), :],
                    dst_ref=out_ref.at[pl.ds(origin_l * m_per, m_per), :],
                    send_sem=send_sems.at[n_right + h],
                    recv_sem=recv_sems.at[n_right + h],
                    device_id=(left,),
                    device_id_type=pl.DeviceIdType.MESH,
                )
                rdma_l.start()
                rdma_l.wait()
            rdma_r.wait()

    return pl.pallas_call(
        body,
        out_shape=jax.ShapeDtypeStruct((N_DEV * m_per, n), jnp.bfloat16),
        grid_spec=pltpu.PrefetchScalarGridSpec(
            num_scalar_prefetch=2,
            in_specs=[pl.BlockSpec(memory_space=pltpu.VMEM)],
            out_specs=pl.BlockSpec(memory_space=pltpu.MemorySpace.HBM),
            scratch_shapes=[
                pltpu.VMEM((m_per, n), jnp.bfloat16),
                pltpu.SemaphoreType.DMA,
                pltpu.SemaphoreType.DMA((N_DEV - 1,)),
                pltpu.SemaphoreType.DMA((N_DEV - 1,)),
            ],
        ),
        compiler_params=pltpu.CompilerParams(collective_id=0),
    )(perm, inv, x)


# device time: 407212 ns/iter; 1.9930x vs baseline; 1.0988x over previous
import jax
import jax.numpy as jnp
from jax import lax
from jax.experimental import pallas as pl
from jax.experimental.pallas import tpu as pltpu

N_DEV = 32


_YZ_PATH = [
    (0, 0), (1, 0), (2, 0), (3, 0),
    (3, 1), (2, 1), (1, 1), (0, 1),
    (0, 2), (1, 2), (2, 2), (3, 2),
    (3, 3), (2, 3), (1, 3), (0, 3),
]

_RING_COORDS = [(0, y, z) for (y, z) in _YZ_PATH] + [
    (1, y, z) for (y, z) in reversed(_YZ_PATH)
]


def _mesh_id(c):
    x, y, z = c
    xi = x if y % 2 == 0 else 1 - x
    return z * 8 + y * 2 + xi


_PERM = [_mesh_id(c) for c in _RING_COORDS]
_INV = [0] * N_DEV
for _r, _m in enumerate(_PERM):
    _INV[_m] = _r
assert sorted(_PERM) == list(range(N_DEV))


def kernel(x):
    m_per, n = x.shape

    perm = jnp.array(_PERM, dtype=jnp.int32)
    inv = jnp.array(_INV, dtype=jnp.int32)

    def body(perm_ref, inv_ref, x_ref, out_ref, own_ref, local_sem,
             send_sems, recv_sems):
        my_pos = lax.axis_index("i")
        r = inv_ref[my_pos]
        right = perm_ref[(r + 1) % N_DEV]
        left = perm_ref[(r - 1) % N_DEV]

        barrier_sem = pltpu.get_barrier_semaphore()
        for nbr in (left, right):
            pl.semaphore_signal(
                barrier_sem, inc=1,
                device_id=(nbr,), device_id_type=pl.DeviceIdType.MESH,
            )
        pl.semaphore_wait(barrier_sem, 2)

        own_ref[:, :] = x_ref[:, :].astype(jnp.bfloat16)
        cp = pltpu.make_async_copy(
            own_ref, out_ref.at[pl.ds(my_pos * m_per, m_per), :], local_sem
        )
        cp.start()
        cp.wait()

        n_hops = N_DEV // 2
        sub = m_per // 2

        def make(dir_idx, target, origin, row_off, k):
            base = origin * m_per + row_off
            return pltpu.make_async_remote_copy(
                src_ref=out_ref.at[pl.ds(base, sub), :],
                dst_ref=out_ref.at[pl.ds(base, sub), :],
                send_sem=send_sems.at[dir_idx, k],
                recv_sem=recv_sems.at[dir_idx, k],
                device_id=(target,),
                device_id_type=pl.DeviceIdType.MESH,
            )

        targets = (right, left)
        rdmas = {}
        recv_waited = set()

        def hop_subs(dir_idx, h):
            if h < n_hops - 1:
                return [(s, s * sub, 2 * h + s) for s in (0, 1)]
            s = 0 if dir_idx == 0 else 1
            return [(s, s * sub, 2 * h)]

        for h in range(n_hops):
            for dir_idx in (0, 1):
                origin = perm_ref[
                    ((r - h) if dir_idx == 0 else (r + h)) % N_DEV
                ]
                for s, row_off, k in hop_subs(dir_idx, h):
                    if h > 0:
                        dep = rdmas[(dir_idx, h - 1, s)]
                        dep.wait_recv()
                        recv_waited.add((dir_idx, h - 1, s))
                    m = make(dir_idx, targets[dir_idx], origin, row_off, k)
                    m.start()
                    rdmas[(dir_idx, h, s)] = m

        for key, m in rdmas.items():
            if key not in recv_waited:
                m.wait_recv()
        for m in rdmas.values():
            m.wait_send()

    return pl.pallas_call(
        body,
        out_shape=jax.ShapeDtypeStruct((N_DEV * m_per, n), jnp.bfloat16),
        grid_spec=pltpu.PrefetchScalarGridSpec(
            num_scalar_prefetch=2,
            in_specs=[pl.BlockSpec(memory_space=pltpu.VMEM)],
            out_specs=pl.BlockSpec(memory_space=pltpu.MemorySpace.HBM),
            scratch_shapes=[
                pltpu.VMEM((m_per, n), jnp.bfloat16),
                pltpu.SemaphoreType.DMA,
                pltpu.SemaphoreType.DMA((2, N_DEV - 1)),
                pltpu.SemaphoreType.DMA((2, N_DEV - 1)),
            ],
        ),
        compiler_params=pltpu.CompilerParams(collective_id=0),
    )(perm, inv, x)


# device time: 405938 ns/iter; 1.9993x vs baseline; 1.0031x over previous
import jax
import jax.numpy as jnp
from jax import lax
from jax.experimental import pallas as pl
from jax.experimental.pallas import tpu as pltpu

N_DEV = 32


_YZ_PATH = [
    (0, 0), (1, 0), (2, 0), (3, 0),
    (3, 1), (2, 1), (1, 1), (0, 1),
    (0, 2), (1, 2), (2, 2), (3, 2),
    (3, 3), (2, 3), (1, 3), (0, 3),
]

_RING_COORDS = [(0, y, z) for (y, z) in _YZ_PATH] + [
    (1, y, z) for (y, z) in reversed(_YZ_PATH)
]


def _mesh_id(c):
    x, y, z = c
    xi = x if y % 2 == 0 else 1 - x
    return z * 8 + y * 2 + xi


_PERM = [_mesh_id(c) for c in _RING_COORDS]
_INV = [0] * N_DEV
for _r, _m in enumerate(_PERM):
    _INV[_m] = _r
assert sorted(_PERM) == list(range(N_DEV))


def kernel(x):
    m_per, n = x.shape

    perm = jnp.array(_PERM, dtype=jnp.int32)
    inv = jnp.array(_INV, dtype=jnp.int32)

    def body(perm_ref, inv_ref, x_ref, out_ref, own_ref, local_sem,
             send_sems, recv_sems):
        my_pos = lax.axis_index("i")
        r = inv_ref[my_pos]
        right = perm_ref[(r + 1) % N_DEV]
        left = perm_ref[(r - 1) % N_DEV]

        barrier_sem = pltpu.get_barrier_semaphore()
        for nbr in (left, right):
            pl.semaphore_signal(
                barrier_sem, inc=1,
                device_id=(nbr,), device_id_type=pl.DeviceIdType.MESH,
            )
        pl.semaphore_wait(barrier_sem, 2)

        own_ref[:, :] = x_ref[:, :].astype(jnp.bfloat16)
        cp = pltpu.make_async_copy(
            own_ref, out_ref.at[pl.ds(my_pos * m_per, m_per), :], local_sem
        )
        cp.start()
        cp.wait()

        n_hops = N_DEV // 2
        n_sub = 4
        sub = m_per // n_sub

        def make(dir_idx, target, origin, row_off, k):
            base = origin * m_per + row_off
            return pltpu.make_async_remote_copy(
                src_ref=out_ref.at[pl.ds(base, sub), :],
                dst_ref=out_ref.at[pl.ds(base, sub), :],
                send_sem=send_sems.at[dir_idx, k],
                recv_sem=recv_sems.at[dir_idx, k],
                device_id=(target,),
                device_id_type=pl.DeviceIdType.MESH,
            )

        targets = (right, left)
        rdmas = {}
        recv_waited = set()

        def hop_subs(dir_idx, h):
            if h < n_hops - 1:
                subs = range(n_sub)
            else:
                subs = (
                    range(n_sub // 2)
                    if dir_idx == 0
                    else range(n_sub // 2, n_sub)
                )
            return [(s, s * sub, n_sub * h + s) for s in subs]

        for h in range(n_hops):
            for dir_idx in (0, 1):
                origin = perm_ref[
                    ((r - h) if dir_idx == 0 else (r + h)) % N_DEV
                ]
                for s, row_off, k in hop_subs(dir_idx, h):
                    if h > 0:
                        dep = rdmas[(dir_idx, h - 1, s)]
                        dep.wait_recv()
                        recv_waited.add((dir_idx, h - 1, s))
                    m = make(dir_idx, targets[dir_idx], origin, row_off, k)
                    m.start()
                    rdmas[(dir_idx, h, s)] = m

        for key, m in rdmas.items():
            if key not in recv_waited:
                m.wait_recv()
        for m in rdmas.values():
            m.wait_send()

    return pl.pallas_call(
        body,
        out_shape=jax.ShapeDtypeStruct((N_DEV * m_per, n), jnp.bfloat16),
        grid_spec=pltpu.PrefetchScalarGridSpec(
            num_scalar_prefetch=2,
            in_specs=[pl.BlockSpec(memory_space=pltpu.VMEM)],
            out_specs=pl.BlockSpec(memory_space=pltpu.MemorySpace.HBM),
            scratch_shapes=[
                pltpu.VMEM((m_per, n), jnp.bfloat16),
                pltpu.SemaphoreType.DMA,
                pltpu.SemaphoreType.DMA((2, 16 * 4)),
                pltpu.SemaphoreType.DMA((2, 16 * 4)),
            ],
        ),
        compiler_params=pltpu.CompilerParams(collective_id=0),
    )(perm, inv, x)
